# baseline (speedup 1.0000x reference)
import functools

import numpy as np

import jax
import jax.numpy as jnp
from jax import lax
from jax.experimental import pallas as pl
from jax.experimental.pallas import tpu as pltpu
from jax.experimental.pallas import tpu_sc as plsc

B = 65536
R = 128
DATA_DIM = 4
STEP = 0.01
BG = 1.0
NSTEP = 174

NC = 2
NS = 16
NW = NC * NS
RPW = B // NW
NG = RPW // 16
NIDX = RPW * 2
NCH = NIDX // 128
GPC = 4

_HI = np.uint32(0xFFFF0000)

_I16 = lambda: lax.iota(jnp.int32, 16)


def _render_body(tab_hbm, ox_h, oy_h, oz_h, dx_h, dy_h, dz_h, tn_h, tx_h,
                 perm_hbm, oidx_hbm, out_hbm,
                 soab, accb, diffb, nchb, permb, oidxb,
                 idxb0, idxb1, valsb0, valsb1, outb,
                 semin, sem0, sem1):
    wid = lax.axis_index("s") * NC + lax.axis_index("c")
    rbase = wid * RPW

    pltpu.sync_copy(perm_hbm.at[pl.ds(rbase, RPW)], permb)
    pltpu.sync_copy(oidx_hbm.at[wid], oidxb)
    comps = (ox_h, oy_h, oz_h, dx_h, dy_h, dz_h, tn_h, tx_h)
    for comp in range(8):
        for c in range(RPW // 128):
            pltpu.async_copy(
                comps[comp].at[permb.at[pl.ds(c * 128, 128)]],
                soab.at[pl.ds(comp * RPW + c * 128, 128)], semin)
    pltpu.make_async_copy(ox_h.at[pl.ds(0, 8 * RPW)], soab, semin).wait()

    def init(g, _):
        base = g * 16
        accb[pl.ds(base, 16)] = jnp.full((16,), 1.0, jnp.float32)
        for ch in range(3):
            accb[pl.ds((1 + ch) * RPW + base, 16)] = jnp.zeros(
                (16,), jnp.float32)
        diffb[pl.ds(base, 16)] = (soab[pl.ds(7 * RPW + base, 16)]
                                  - soab[pl.ds(6 * RPW + base, 16)])
        return 0

    lax.fori_loop(0, NG, init, 0)

    for half in range(2):
        acc = jnp.full((16,), -1.0, jnp.float32)
        for j in range(64):
            v = plsc.load_gather(diffb, [half * 1024 + _I16() * 64 + j])
            acc = jnp.maximum(acc, v)
        n = (acc * (1.0 / STEP)).astype(jnp.int32) + 2
        nchb[pl.ds(half * 16, 16)] = jnp.clip(n, 0, NSTEP)

    def alive(s):
        lo = nchb[pl.ds(0, 16)]
        hi = nchb[pl.ds(16, 16)]
        c0 = plsc.all_reduce_population_count(lo > s)
        c1 = plsc.all_reduce_population_count(hi > s)
        return jnp.max(c0) + jnp.max(c1)

    n_tile = jnp.max(jnp.maximum(nchb[pl.ds(0, 16)], nchb[pl.ds(16, 16)]))

    def compute_idx_group(g, ts, idxb):
        base = g * 16
        t = soab[pl.ds(6 * RPW + base, 16)] + ts
        px = soab[pl.ds(0 * RPW + base, 16)] + t * soab[
            pl.ds(3 * RPW + base, 16)]
        py = soab[pl.ds(1 * RPW + base, 16)] + t * soab[
            pl.ds(4 * RPW + base, 16)]
        pz = soab[pl.ds(2 * RPW + base, 16)] + t * soab[
            pl.ds(5 * RPW + base, 16)]
        ix = jnp.clip(px, 0.0, 127.0).astype(jnp.int32)
        iy = jnp.clip(py, 0.0, 127.0).astype(jnp.int32)
        iz = jnp.clip(pz, 0.0, 127.0).astype(jnp.int32)
        lin2 = ((ix * R + iy) * R + iz) * 2
        b2 = base * 2
        idxb[pl.ds(b2, 16)] = lin2
        idxb[pl.ds(b2 + 16, 16)] = lin2 + 1

    def issue(idxb, valsb, sem, m):
        for c in range(NCH):
            @pl.when(c < m)
            def _():
                pltpu.async_copy(
                    tab_hbm.at[idxb.at[pl.ds(c * 128, 128)]],
                    valsb.at[pl.ds(c * 128, 128)], sem)

    def drain(valsb, sem, m):
        for c in range(NCH):
            @pl.when(c < m)
            def _():
                pltpu.make_async_copy(
                    tab_hbm.at[pl.ds(0, 128)],
                    valsb.at[pl.ds(c * 128, 128)], sem).wait()

    def phase(s, valsb, idxb, sem, m_s, m_s2):
        drain(valsb, sem, m_s)
        ts = (s.astype(jnp.float32) + 0.5) * STEP
        ts2 = ts + 2.0 * STEP

        def body(g, _):
            base = g * 16
            b2 = base * 2
            t = soab[pl.ds(6 * RPW + base, 16)] + ts
            tmx = soab[pl.ds(7 * RPW + base, 16)]
            valid = t < tmx
            w0 = valsb[pl.ds(b2, 16)]
            w1 = valsb[pl.ds(b2 + 16, 16)]
            sig = plsc.bitcast(w1 & _HI, jnp.float32)
            sig = jnp.maximum(sig, 0.0)
            att = jnp.exp(sig * (-STEP))
            att = jnp.where(valid, att, 1.0)
            lgt = accb[pl.ds(base, 16)]
            w = lgt * (1.0 - att)
            chans = (plsc.bitcast(w0 << 16, jnp.float32),
                     plsc.bitcast(w0 & _HI, jnp.float32),
                     plsc.bitcast(w1 << 16, jnp.float32))
            for ch in range(3):
                rgb = 1.0 / (1.0 + jnp.exp(-chans[ch]))
                off = (1 + ch) * RPW + base
                accb[pl.ds(off, 16)] = accb[pl.ds(off, 16)] + w * rgb
            accb[pl.ds(base, 16)] = lgt * att
            compute_idx_group(g, ts2, idxb)
            return 0

        lax.fori_loop(0, m_s * GPC, body, 0)
        issue(idxb, valsb, sem, m_s2)

    m0 = alive(jnp.int32(0))
    m1 = alive(jnp.int32(1))

    def pro0(g, _):
        compute_idx_group(g, jnp.float32(0.5 * STEP), idxb0)
        return 0

    def pro1(g, _):
        compute_idx_group(g, jnp.float32(1.5 * STEP), idxb1)
        return 0

    lax.fori_loop(0, m0 * GPC, pro0, 0)
    issue(idxb0, valsb0, sem0, m0)
    lax.fori_loop(0, m1 * GPC, pro1, 0)
    issue(idxb1, valsb1, sem1, m1)

    def two_steps(k, _):
        s0 = 2 * k
        phase(s0, valsb0, idxb0, sem0, alive(s0), alive(s0 + 2))
        phase(s0 + 1, valsb1, idxb1, sem1, alive(s0 + 1), alive(s0 + 3))
        return 0

    lax.fori_loop(0, (n_tile + 1) // 2, two_steps, 0)

    def fin(g, _):
        base = g * 16
        b4 = base * 4
        lgt = accb[pl.ds(base, 16)]
        for ch in range(3):
            val = accb[pl.ds((1 + ch) * RPW + base, 16)] + lgt * BG
            outb[pl.ds(b4 + ch * 16, 16)] = val
        outb[pl.ds(b4 + 48, 16)] = lgt
        return 0

    lax.fori_loop(0, NG, fin, 0)
    for c in range(RPW * 4 // 128):
        pltpu.async_copy(outb.at[pl.ds(c * 128, 128)],
                         out_hbm.at[oidxb.at[c]], semin)
    pltpu.make_async_copy(ox_h.at[pl.ds(0, RPW * 4)], outb, semin).wait()


_mesh = plsc.VectorSubcoreMesh(core_axis_name="c", subcore_axis_name="s")

_render = functools.partial(
    pl.kernel,
    out_type=jax.ShapeDtypeStruct((B * DATA_DIM,), jnp.float32),
    mesh=_mesh,
    compiler_params=pltpu.CompilerParams(needs_layout_passes=False),
    scratch_types=[
        pltpu.VMEM((8 * RPW,), jnp.float32),
        pltpu.VMEM((4 * RPW,), jnp.float32),
        pltpu.VMEM((RPW,), jnp.float32),
        pltpu.VMEM((32,), jnp.int32),
        pltpu.VMEM((RPW,), jnp.int32),
        pltpu.VMEM((RPW * 4 // 128, 128), jnp.int32),
        pltpu.VMEM((NIDX,), jnp.int32),
        pltpu.VMEM((NIDX,), jnp.int32),
        pltpu.VMEM((NIDX,), jnp.uint32),
        pltpu.VMEM((NIDX,), jnp.uint32),
        pltpu.VMEM((RPW * 4,), jnp.float32),
        pltpu.SemaphoreType.DMA,
        pltpu.SemaphoreType.DMA,
        pltpu.SemaphoreType.DMA,
    ],
)(_render_body)


def kernel(data, origins, dirs, viewdirs):
    del viewdirs
    dirs = dirs / jnp.linalg.norm(dirs, axis=-1, keepdims=True)
    eps = 1e-9
    safe = jnp.where(jnp.abs(dirs) > eps, dirs,
                     jnp.where(dirs >= 0, eps, -eps))
    invdir = 1.0 / safe
    t1 = -origins * invdir
    t2 = t1 + invdir
    tmin = jnp.maximum(jnp.max(jnp.minimum(t1, t2), axis=-1), 0.0)
    tmax = jnp.min(jnp.maximum(t1, t2), axis=-1)
    o128 = origins * R
    d128 = dirs * R

    order = jnp.argsort(tmin - tmax).astype(jnp.int32)
    perm = order.reshape(RPW, NW).T.reshape(B)
    pw = perm.reshape(NW, NG, 16)
    oidx = (pw[:, :, None, :] * 4
            + jnp.arange(4, dtype=jnp.int32)[None, None, :, None])
    oidx = oidx.reshape(NW, RPW * 4 // 128, 128)

    bf = data.astype(jnp.bfloat16).reshape(R * R * R, 2, 2)
    tab = lax.bitcast_convert_type(bf, jnp.uint32).reshape(R * R * R * 2)

    out_flat = _render(tab, o128[:, 0], o128[:, 1], o128[:, 2],
                       d128[:, 0], d128[:, 1], d128[:, 2],
                       tmin, tmax, perm, oidx)
    return out_flat.reshape(B, DATA_DIM)[:, :3]


# speedup vs baseline: 2.4832x; 2.4832x over previous
import functools

import numpy as np

import jax
import jax.numpy as jnp
from jax import lax
from jax.experimental import pallas as pl
from jax.experimental.pallas import tpu as pltpu
from jax.experimental.pallas import tpu_sc as plsc

B = 65536
R = 128
DATA_DIM = 4
STEP = 0.01
BG = 1.0
NSTEP = 174

NC = 2
NS = 16
NW = NC * NS
RPW = B // NW
NG = RPW // 16
NIDX = RPW * 2
NCH = NIDX // 128
GPC = 4

_HI = np.uint32(0xFFFF0000)

_I16 = lambda: lax.iota(jnp.int32, 16)


def _render_body(tab_hbm, ox_h, oy_h, oz_h, dx_h, dy_h, dz_h, tn_h, tx_h,
                 perm_hbm, oidx_hbm, out_hbm,
                 soab, accb, diffb, nchb, permb, oidxb,
                 idxb0, idxb1, valsb0, valsb1, outb,
                 semin, sem0, sem1):
    wid = lax.axis_index("s") * NC + lax.axis_index("c")
    rbase = wid * RPW

    pltpu.sync_copy(perm_hbm.at[pl.ds(rbase, RPW)], permb)
    pltpu.sync_copy(oidx_hbm.at[wid], oidxb)
    comps = (ox_h, oy_h, oz_h, dx_h, dy_h, dz_h, tn_h, tx_h)
    for comp in range(8):
        for c in range(RPW // 128):
            pltpu.async_copy(
                comps[comp].at[permb.at[pl.ds(c * 128, 128)]],
                soab.at[pl.ds(comp * RPW + c * 128, 128)], semin)
    pltpu.make_async_copy(ox_h.at[pl.ds(0, 8 * RPW)], soab, semin).wait()

    def init(g, _):
        base = g * 16
        accb[pl.ds(base, 16)] = jnp.full((16,), 1.0, jnp.float32)
        for ch in range(3):
            accb[pl.ds((1 + ch) * RPW + base, 16)] = jnp.zeros(
                (16,), jnp.float32)
        diffb[pl.ds(base, 16)] = (soab[pl.ds(7 * RPW + base, 16)]
                                  - soab[pl.ds(6 * RPW + base, 16)])
        return 0

    lax.fori_loop(0, NG, init, 0)

    for half in range(2):
        acc = jnp.full((16,), -1.0, jnp.float32)
        for j in range(64):
            v = plsc.load_gather(diffb, [half * 1024 + _I16() * 64 + j])
            acc = jnp.maximum(acc, v)
        n = (acc * (1.0 / STEP)).astype(jnp.int32) + 2
        nchb[pl.ds(half * 16, 16)] = jnp.clip(n, 0, NSTEP)

    def alive(s):
        lo = nchb[pl.ds(0, 16)]
        hi = nchb[pl.ds(16, 16)]
        c0 = plsc.all_reduce_population_count(lo > s)
        c1 = plsc.all_reduce_population_count(hi > s)
        return jnp.max(c0) + jnp.max(c1)

    n_tile = jnp.max(jnp.maximum(nchb[pl.ds(0, 16)], nchb[pl.ds(16, 16)]))

    def compute_idx_group(g, ts, idxb):
        base = g * 16
        t = soab[pl.ds(6 * RPW + base, 16)] + ts
        px = soab[pl.ds(0 * RPW + base, 16)] + t * soab[
            pl.ds(3 * RPW + base, 16)]
        py = soab[pl.ds(1 * RPW + base, 16)] + t * soab[
            pl.ds(4 * RPW + base, 16)]
        pz = soab[pl.ds(2 * RPW + base, 16)] + t * soab[
            pl.ds(5 * RPW + base, 16)]
        ix = jnp.clip(px, 0.0, 127.0).astype(jnp.int32)
        iy = jnp.clip(py, 0.0, 127.0).astype(jnp.int32)
        iz = jnp.clip(pz, 0.0, 127.0).astype(jnp.int32)
        lin2 = (ix * R + iy) * (2 * R) + iz
        b2 = base * 2
        idxb[pl.ds(b2, 16)] = lin2
        idxb[pl.ds(b2 + 16, 16)] = lin2 + R

    def issue(idxb, valsb, sem, m):
        for c in range(NCH):
            @pl.when(c < m)
            def _():
                pltpu.async_copy(
                    tab_hbm.at[idxb.at[pl.ds(c * 128, 128)]],
                    valsb.at[pl.ds(c * 128, 128)], sem)

    def drain(valsb, sem, m):
        for c in range(NCH):
            @pl.when(c < m)
            def _():
                pltpu.make_async_copy(
                    tab_hbm.at[pl.ds(0, 128)],
                    valsb.at[pl.ds(c * 128, 128)], sem).wait()

    def phase(s, valsb, idxb, sem, m_s, m_s2):
        drain(valsb, sem, m_s)
        ts = (s.astype(jnp.float32) + 0.5) * STEP
        ts2 = ts + 2.0 * STEP

        def body(g, _):
            base = g * 16
            b2 = base * 2
            t = soab[pl.ds(6 * RPW + base, 16)] + ts
            tmx = soab[pl.ds(7 * RPW + base, 16)]
            valid = t < tmx
            w0 = valsb[pl.ds(b2, 16)]
            w1 = valsb[pl.ds(b2 + 16, 16)]
            sig = plsc.bitcast(w1 & _HI, jnp.float32)
            sig = jnp.maximum(sig, 0.0)
            att = jnp.exp(sig * (-STEP))
            att = jnp.where(valid, att, 1.0)
            lgt = accb[pl.ds(base, 16)]
            w = lgt * (1.0 - att)
            chans = (plsc.bitcast(w0 << 16, jnp.float32),
                     plsc.bitcast(w0 & _HI, jnp.float32),
                     plsc.bitcast(w1 << 16, jnp.float32))
            for ch in range(3):
                rgb = 1.0 / (1.0 + jnp.exp(-chans[ch]))
                off = (1 + ch) * RPW + base
                accb[pl.ds(off, 16)] = accb[pl.ds(off, 16)] + w * rgb
            accb[pl.ds(base, 16)] = lgt * att
            compute_idx_group(g, ts2, idxb)
            return 0

        lax.fori_loop(0, m_s * GPC, body, 0)
        issue(idxb, valsb, sem, m_s2)

    m0 = alive(jnp.int32(0))
    m1 = alive(jnp.int32(1))

    def pro0(g, _):
        compute_idx_group(g, jnp.float32(0.5 * STEP), idxb0)
        return 0

    def pro1(g, _):
        compute_idx_group(g, jnp.float32(1.5 * STEP), idxb1)
        return 0

    lax.fori_loop(0, m0 * GPC, pro0, 0)
    issue(idxb0, valsb0, sem0, m0)
    lax.fori_loop(0, m1 * GPC, pro1, 0)
    issue(idxb1, valsb1, sem1, m1)

    def two_steps(k, _):
        s0 = 2 * k
        phase(s0, valsb0, idxb0, sem0, alive(s0), alive(s0 + 2))
        phase(s0 + 1, valsb1, idxb1, sem1, alive(s0 + 1), alive(s0 + 3))
        return 0

    lax.fori_loop(0, (n_tile + 1) // 2, two_steps, 0)

    def fin(g, _):
        base = g * 16
        b4 = base * 4
        lgt = accb[pl.ds(base, 16)]
        for ch in range(3):
            val = accb[pl.ds((1 + ch) * RPW + base, 16)] + lgt * BG
            outb[pl.ds(b4 + ch * 16, 16)] = val
        outb[pl.ds(b4 + 48, 16)] = lgt
        return 0

    lax.fori_loop(0, NG, fin, 0)
    for c in range(RPW * 4 // 128):
        pltpu.async_copy(outb.at[pl.ds(c * 128, 128)],
                         out_hbm.at[oidxb.at[c]], semin)
    pltpu.make_async_copy(ox_h.at[pl.ds(0, RPW * 4)], outb, semin).wait()


_mesh = plsc.VectorSubcoreMesh(core_axis_name="c", subcore_axis_name="s")

_render = functools.partial(
    pl.kernel,
    out_type=jax.ShapeDtypeStruct((B * DATA_DIM,), jnp.float32),
    mesh=_mesh,
    compiler_params=pltpu.CompilerParams(needs_layout_passes=False),
    scratch_types=[
        pltpu.VMEM((8 * RPW,), jnp.float32),
        pltpu.VMEM((4 * RPW,), jnp.float32),
        pltpu.VMEM((RPW,), jnp.float32),
        pltpu.VMEM((32,), jnp.int32),
        pltpu.VMEM((RPW,), jnp.int32),
        pltpu.VMEM((RPW * 4 // 128, 128), jnp.int32),
        pltpu.VMEM((NIDX,), jnp.int32),
        pltpu.VMEM((NIDX,), jnp.int32),
        pltpu.VMEM((NIDX,), jnp.uint32),
        pltpu.VMEM((NIDX,), jnp.uint32),
        pltpu.VMEM((RPW * 4,), jnp.float32),
        pltpu.SemaphoreType.DMA,
        pltpu.SemaphoreType.DMA,
        pltpu.SemaphoreType.DMA,
    ],
)(_render_body)


def kernel(data, origins, dirs, viewdirs):
    del viewdirs
    dirs = dirs / jnp.linalg.norm(dirs, axis=-1, keepdims=True)
    eps = 1e-9
    safe = jnp.where(jnp.abs(dirs) > eps, dirs,
                     jnp.where(dirs >= 0, eps, -eps))
    invdir = 1.0 / safe
    t1 = -origins * invdir
    t2 = t1 + invdir
    tmin = jnp.maximum(jnp.max(jnp.minimum(t1, t2), axis=-1), 0.0)
    tmax = jnp.min(jnp.maximum(t1, t2), axis=-1)
    o128 = origins * R
    d128 = dirs * R

    order = jnp.argsort(tmin - tmax).astype(jnp.int32)
    perm = order.reshape(RPW, NW).T.reshape(B)
    pw = perm.reshape(NW, NG, 16)
    oidx = (pw[:, :, None, :] * 4
            + jnp.arange(4, dtype=jnp.int32)[None, None, :, None])
    oidx = oidx.reshape(NW, RPW * 4 // 128, 128)

    bits = lax.bitcast_convert_type(
        data.astype(jnp.bfloat16), jnp.uint16).astype(jnp.uint32)
    lo = bits[:, :, :, 0::2]
    hi = bits[:, :, :, 1::2]
    wrd = jnp.transpose(lo | (hi << 16), (0, 1, 3, 2))
    tab = wrd.reshape(R * R * R * 2)

    out_flat = _render(tab, o128[:, 0], o128[:, 1], o128[:, 2],
                       d128[:, 0], d128[:, 1], d128[:, 2],
                       tmin, tmax, perm, oidx)
    return out_flat.reshape(B, DATA_DIM)[:, :3]


# speedup vs baseline: 4.1384x; 1.6666x over previous
import functools

import numpy as np

import jax
import jax.numpy as jnp
from jax import lax
from jax.experimental import pallas as pl
from jax.experimental.pallas import tpu as pltpu
from jax.experimental.pallas import tpu_sc as plsc

B = 65536
R = 128
DATA_DIM = 4
STEP = 0.01
BG = 1.0
NSTEP = 174

NC = 2
NS = 16
NW = NC * NS
RPW = B // NW
NG = RPW // 16
NIDX = RPW * 2
NCH = NIDX // 128
GPC = 4

_HI = np.uint32(0xFFFF0000)

_I16 = lambda: lax.iota(jnp.int32, 16)


def _render_body(tab_hbm, ox_h, oy_h, oz_h, dx_h, dy_h, dz_h, tn_h, tx_h,
                 perm_hbm, oidx_hbm, out_hbm,
                 soab, accb, diffb, nchb, permb, oidxb,
                 idxb0, idxb1, valsb0, valsb1, outb,
                 semin, sem0, sem1):
    wid = lax.axis_index("s") * NC + lax.axis_index("c")
    rbase = wid * RPW

    pltpu.sync_copy(perm_hbm.at[pl.ds(rbase, RPW)], permb)
    pltpu.sync_copy(oidx_hbm.at[wid], oidxb)
    comps = (ox_h, oy_h, oz_h, dx_h, dy_h, dz_h, tn_h, tx_h)
    for comp in range(8):
        for c in range(RPW // 128):
            pltpu.async_copy(
                comps[comp].at[permb.at[pl.ds(c * 128, 128)]],
                soab.at[pl.ds(comp * RPW + c * 128, 128)], semin)
    pltpu.make_async_copy(ox_h.at[pl.ds(0, 8 * RPW)], soab, semin).wait()

    def init(g, _):
        base = g * 16
        accb[pl.ds(base, 16)] = jnp.full((16,), 1.0, jnp.float32)
        for ch in range(3):
            accb[pl.ds((1 + ch) * RPW + base, 16)] = jnp.zeros(
                (16,), jnp.float32)
        diffb[pl.ds(base, 16)] = (soab[pl.ds(7 * RPW + base, 16)]
                                  - soab[pl.ds(6 * RPW + base, 16)])
        return 0

    lax.fori_loop(0, NG, init, 0)

    for half in range(2):
        acc = jnp.full((16,), -1.0, jnp.float32)
        for j in range(64):
            v = plsc.load_gather(diffb, [half * 1024 + _I16() * 64 + j])
            acc = jnp.maximum(acc, v)
        n = (acc * (1.0 / STEP)).astype(jnp.int32) + 2
        nchb[pl.ds(half * 16, 16)] = jnp.clip(n, 0, NSTEP)

    def alive(s):
        lo = nchb[pl.ds(0, 16)]
        hi = nchb[pl.ds(16, 16)]
        c0 = plsc.all_reduce_population_count(lo > s)
        c1 = plsc.all_reduce_population_count(hi > s)
        return jnp.max(c0) + jnp.max(c1)

    n_tile = jnp.max(jnp.maximum(nchb[pl.ds(0, 16)], nchb[pl.ds(16, 16)]))

    def compute_idx_group(g, ts, idxb):
        base = g * 16
        t = soab[pl.ds(6 * RPW + base, 16)] + ts
        px = soab[pl.ds(0 * RPW + base, 16)] + t * soab[
            pl.ds(3 * RPW + base, 16)]
        py = soab[pl.ds(1 * RPW + base, 16)] + t * soab[
            pl.ds(4 * RPW + base, 16)]
        pz = soab[pl.ds(2 * RPW + base, 16)] + t * soab[
            pl.ds(5 * RPW + base, 16)]
        ix = jnp.clip(px, 0.0, 127.0).astype(jnp.int32)
        iy = jnp.clip(py, 0.0, 127.0).astype(jnp.int32)
        iz = jnp.clip(pz, 0.0, 127.0).astype(jnp.int32)
        lin2 = (ix * R + iy) * (2 * R) + iz
        b2 = base * 2
        idxb[pl.ds(b2, 16)] = lin2
        idxb[pl.ds(b2 + 16, 16)] = lin2 + R

    def issue(idxb, valsb, sem, m):
        for c in range(NCH):
            @pl.when(c < m)
            def _():
                pltpu.async_copy(
                    tab_hbm.at[idxb.at[pl.ds(c * 128, 128)]],
                    valsb.at[pl.ds(c * 128, 128)], sem)

    def drain(valsb, sem, m):
        for c in range(NCH):
            @pl.when(c < m)
            def _():
                pltpu.make_async_copy(
                    tab_hbm.at[pl.ds(0, 128)],
                    valsb.at[pl.ds(c * 128, 128)], sem).wait()

    def phase(s, valsb, idxb, sem, m_s, m_s2):
        drain(valsb, sem, m_s)
        ts = (s.astype(jnp.float32) + 0.5) * STEP
        ts2 = ts + 2.0 * STEP

        def body(g, _):
            base = g * 16
            b2 = base * 2
            t = soab[pl.ds(6 * RPW + base, 16)] + ts
            tmx = soab[pl.ds(7 * RPW + base, 16)]
            valid = t < tmx
            w0 = valsb[pl.ds(b2, 16)]
            w1 = valsb[pl.ds(b2 + 16, 16)]
            sig = plsc.bitcast(w1 & _HI, jnp.float32)
            sig = jnp.maximum(sig, 0.0)
            att = jnp.exp(sig * (-STEP))
            att = jnp.where(valid, att, 1.0)
            lgt = accb[pl.ds(base, 16)]
            w = lgt * (1.0 - att)
            chans = (plsc.bitcast(w0 << 16, jnp.float32),
                     plsc.bitcast(w0 & _HI, jnp.float32),
                     plsc.bitcast(w1 << 16, jnp.float32))
            for ch in range(3):
                rgb = 1.0 / (1.0 + jnp.exp(-chans[ch]))
                off = (1 + ch) * RPW + base
                accb[pl.ds(off, 16)] = accb[pl.ds(off, 16)] + w * rgb
            accb[pl.ds(base, 16)] = lgt * att
            compute_idx_group(g, ts2, idxb)
            return 0

        lax.fori_loop(0, m_s * GPC, body, 0)
        issue(idxb, valsb, sem, m_s2)

    m0 = alive(jnp.int32(0))
    m1 = alive(jnp.int32(1))

    def pro0(g, _):
        compute_idx_group(g, jnp.float32(0.5 * STEP), idxb0)
        return 0

    def pro1(g, _):
        compute_idx_group(g, jnp.float32(1.5 * STEP), idxb1)
        return 0

    lax.fori_loop(0, m0 * GPC, pro0, 0)
    issue(idxb0, valsb0, sem0, m0)
    lax.fori_loop(0, m1 * GPC, pro1, 0)
    issue(idxb1, valsb1, sem1, m1)

    def two_steps(k, _):
        s0 = 2 * k
        phase(s0, valsb0, idxb0, sem0, alive(s0), alive(s0 + 2))
        phase(s0 + 1, valsb1, idxb1, sem1, alive(s0 + 1), alive(s0 + 3))
        return 0

    lax.fori_loop(0, (n_tile + 1) // 2, two_steps, 0)

    def fin(g, _):
        base = g * 16
        b4 = base * 4
        lgt = accb[pl.ds(base, 16)]
        for ch in range(3):
            val = accb[pl.ds((1 + ch) * RPW + base, 16)] + lgt * BG
            outb[pl.ds(b4 + ch * 16, 16)] = val
        outb[pl.ds(b4 + 48, 16)] = lgt
        return 0

    lax.fori_loop(0, NG, fin, 0)
    for c in range(RPW * 4 // 128):
        pltpu.async_copy(outb.at[pl.ds(c * 128, 128)],
                         out_hbm.at[oidxb.at[c]], semin)
    pltpu.make_async_copy(ox_h.at[pl.ds(0, RPW * 4)], outb, semin).wait()


_mesh = plsc.VectorSubcoreMesh(core_axis_name="c", subcore_axis_name="s")

_render = functools.partial(
    pl.kernel,
    out_type=jax.ShapeDtypeStruct((B * DATA_DIM,), jnp.float32),
    mesh=_mesh,
    compiler_params=pltpu.CompilerParams(needs_layout_passes=False),
    scratch_types=[
        pltpu.VMEM((8 * RPW,), jnp.float32),
        pltpu.VMEM((4 * RPW,), jnp.float32),
        pltpu.VMEM((RPW,), jnp.float32),
        pltpu.VMEM((32,), jnp.int32),
        pltpu.VMEM((RPW,), jnp.int32),
        pltpu.VMEM((RPW * 4 // 128, 128), jnp.int32),
        pltpu.VMEM((NIDX,), jnp.int32),
        pltpu.VMEM((NIDX,), jnp.int32),
        pltpu.VMEM((NIDX,), jnp.uint32),
        pltpu.VMEM((NIDX,), jnp.uint32),
        pltpu.VMEM((RPW * 4,), jnp.float32),
        pltpu.SemaphoreType.DMA,
        pltpu.SemaphoreType.DMA,
        pltpu.SemaphoreType.DMA,
    ],
)(_render_body)


def kernel(data, origins, dirs, viewdirs):
    del viewdirs
    dirs = dirs / jnp.linalg.norm(dirs, axis=-1, keepdims=True)
    eps = 1e-9
    safe = jnp.where(jnp.abs(dirs) > eps, dirs,
                     jnp.where(dirs >= 0, eps, -eps))
    invdir = 1.0 / safe
    t1 = -origins * invdir
    t2 = t1 + invdir
    tmin = jnp.maximum(jnp.max(jnp.minimum(t1, t2), axis=-1), 0.0)
    tmax = jnp.min(jnp.maximum(t1, t2), axis=-1)
    o128 = origins * R
    d128 = dirs * R

    order = jnp.argsort(tmin - tmax).astype(jnp.int32)
    perm = order.reshape(RPW, NW).T.reshape(B)
    pw = perm.reshape(NW, NG, 16)
    oidx = (pw[:, :, None, :] * 4
            + jnp.arange(4, dtype=jnp.int32)[None, None, :, None])
    oidx = oidx.reshape(NW, RPW * 4 // 128, 128)

    bits = lax.bitcast_convert_type(
        data.astype(jnp.bfloat16), jnp.uint16).astype(jnp.uint32)
    w0 = bits[:, :, :, 0] | (bits[:, :, :, 1] << 16)
    w1 = bits[:, :, :, 2] | (bits[:, :, :, 3] << 16)
    tab = jnp.stack([w0, w1], axis=2).reshape(R * R * R * 2)

    out_flat = _render(tab, o128[:, 0], o128[:, 1], o128[:, 2],
                       d128[:, 0], d128[:, 1], d128[:, 2],
                       tmin, tmax, perm, oidx)
    return out_flat.reshape(B, DATA_DIM)[:, :3]
